# baseline (speedup 1.0000x reference)
import functools

import jax
import jax.numpy as jnp
from jax import lax
from jax.experimental import pallas as pl
from jax.experimental.pallas import tpu as pltpu
from jax.experimental.pallas import tpu_sc as plsc

N = 10000
E = 160000
D_IN = 256
D_OUT = 512
H = 128
W = H + 16
NTILES = 16
CHUNK = 112
GRP = 10
NG = 9
NCH = GRP * NG
EPAD = NTILES * CHUNK * NCH
NP = 10016
RPT = NP // NTILES


def _sc_segment_sums(xaug, src2, dst2):
    mesh = plsc.VectorSubcoreMesh(core_axis_name="c", subcore_axis_name="s")

    @functools.partial(
        pl.kernel,
        out_type=jax.ShapeDtypeStruct((2, NP, W), jnp.float32),
        mesh=mesh,
        compiler_params=pltpu.CompilerParams(use_tc_tiling_on_sc=False),
        scratch_types=[
            pltpu.VMEM((GRP, CHUNK), jnp.int32),
            pltpu.VMEM((GRP, CHUNK), jnp.int32),
            pltpu.VMEM((2, CHUNK, W), jnp.float32),
            pltpu.VMEM_SHARED((NP, W), jnp.float32),
            pltpu.SemaphoreType.DMA,
            pltpu.SemaphoreType.DMA,
            pltpu.SemaphoreType.DMA,
            pltpu.SemaphoreType.DMA,
        ],
    )
    def seg(xaug_hbm, src_hbm, dst_hbm, sums_hbm, srcb, dstb, rows, acc_sh,
            g0, g1, s0, s1):
        c = lax.axis_index("c")
        s = lax.axis_index("s")
        row0 = s * RPT

        zv = jnp.zeros((16,), jnp.float32)

        @pl.loop(0, CHUNK)
        def _(i):
            @pl.loop(0, W // 16)
            def _(k):
                rows[0, i, pl.ds(k * 16, 16)] = zv

        @pl.loop(0, RPT // CHUNK)
        def _(r):
            pltpu.sync_copy(rows.at[0], acc_sh.at[pl.ds(row0 + r * CHUNK, CHUNK)])
        rem = RPT - (RPT // CHUNK) * CHUNK
        if rem:
            pltpu.sync_copy(rows.at[0].at[pl.ds(0, rem)],
                            acc_sh.at[pl.ds(row0 + RPT - rem, rem)])

        plsc.subcore_barrier()

        sbase = (c * NTILES + s) * NCH
        dbase = s * NCH

        def gather_start(k, b, sem):
            pltpu.async_copy(xaug_hbm.at[srcb.at[k]], rows.at[b], sem)

        def gather_wait(k, b, sem):
            pltpu.make_async_copy(xaug_hbm.at[srcb.at[k]], rows.at[b], sem).wait()

        def scatter(k, b):
            pltpu.sync_copy(rows.at[b], acc_sh.at[dstb.at[k]], add=True)

        @pl.loop(0, NG)
        def _(g):
            pltpu.sync_copy(src_hbm.at[pl.ds(sbase + g * GRP, GRP)], srcb)
            pltpu.sync_copy(dst_hbm.at[pl.ds(dbase + g * GRP, GRP)], dstb)
            gather_start(0, 0, g0)

            @pl.loop(0, GRP // 2)
            def _(t):
                k = t * 2
                gather_start(k + 1, 1, g1)
                gather_wait(k, 0, g0)
                scatter(k, 0)

                @pl.when(t < GRP // 2 - 1)
                def _():
                    gather_start(k + 2, 0, g0)

                gather_wait(k + 1, 1, g1)
                scatter(k + 1, 1)

        plsc.subcore_barrier()

        pltpu.sync_copy(acc_sh.at[pl.ds(row0, RPT)],
                        sums_hbm.at[c].at[pl.ds(row0, RPT)])

    return seg(xaug, src2, dst2)


_BN = 2000


def _tc_combine(sums2, xb, wl0, wl1, wr, b):
    def body(s0_ref, s1_ref, x_ref, w0_ref, w1_ref, wr_ref, b_ref, o_ref):
        s0 = s0_ref[0]
        s1 = s1_ref[0]
        r = 1.0 / jnp.maximum(s0[:, H:H + 1], 1.0)
        a0 = (s0[:, :H] * r).astype(jnp.bfloat16)
        a1 = (s1[:, :H] * r).astype(jnp.bfloat16)
        acc = jnp.dot(a0, w0_ref[...], preferred_element_type=jnp.float32)
        acc = acc + jnp.dot(a1, w1_ref[...], preferred_element_type=jnp.float32)
        acc = acc + jnp.dot(x_ref[...], wr_ref[...],
                            preferred_element_type=jnp.float32)
        o_ref[...] = jnp.maximum(acc + b_ref[...], 0.0)

    return pl.pallas_call(
        body,
        grid=(N // _BN,),
        in_specs=[
            pl.BlockSpec((1, _BN, W), lambda i: (0, i, 0)),
            pl.BlockSpec((1, _BN, W), lambda i: (1, i, 0)),
            pl.BlockSpec((_BN, D_IN), lambda i: (i, 0)),
            pl.BlockSpec((H, D_OUT), lambda i: (0, 0)),
            pl.BlockSpec((H, D_OUT), lambda i: (0, 0)),
            pl.BlockSpec((D_IN, D_OUT), lambda i: (0, 0)),
            pl.BlockSpec((1, D_OUT), lambda i: (0, 0)),
        ],
        out_specs=pl.BlockSpec((_BN, D_OUT), lambda i: (i, 0)),
        out_shape=jax.ShapeDtypeStruct((N, D_OUT), jnp.float32),
    )(sums2, sums2, xb, wl0, wl1, wr, b)


def kernel(x, edge_index, W_l, b_l, W_r):
    xh = x.reshape(2 * N, H)
    xaug = jnp.concatenate([xh, jnp.ones((2 * N, W - H), jnp.float32)], axis=1)

    src = edge_index[0]
    dst = edge_index[1]
    pad = EPAD - E
    srcp = jnp.concatenate([src, jnp.zeros((pad,), jnp.int32)])
    dstp = jnp.concatenate([dst, jnp.full((pad,), N, jnp.int32)])
    s2 = srcp * 2
    src2 = jnp.concatenate([s2, s2 + 1]).reshape(2 * NTILES * NCH, CHUNK)
    dst2 = dstp.reshape(NTILES * NCH, CHUNK)

    sums = _sc_segment_sums(xaug, src2, dst2)

    wlt = W_l.T.astype(jnp.bfloat16)
    out = _tc_combine(sums, x.astype(jnp.bfloat16),
                      wlt[:H], wlt[H:], W_r.T.astype(jnp.bfloat16),
                      b_l.reshape(1, D_OUT))
    return out


# speedup vs baseline: 1.2421x; 1.2421x over previous
import functools

import jax
import jax.numpy as jnp
from jax import lax
from jax.experimental import pallas as pl
from jax.experimental.pallas import tpu as pltpu
from jax.experimental.pallas import tpu_sc as plsc

N = 10000
E = 160000
D_IN = 256
D_OUT = 512
H = 128
NTILES = 16
CHUNK = 112
GRP = 10
NG = 9
NCH = GRP * NG
EPAD = NTILES * CHUNK * NCH
NP = 10016
NP2 = 10240
RPT = NP // NTILES


def _sc_segment_sums(xtab, src2, dst2):
    mesh = plsc.VectorSubcoreMesh(core_axis_name="c", subcore_axis_name="s")

    @functools.partial(
        pl.kernel,
        out_type=[
            jax.ShapeDtypeStruct((2, NP, H), jnp.float32),
            jax.ShapeDtypeStruct((NTILES, NP2), jnp.float32),
        ],
        mesh=mesh,
        compiler_params=pltpu.CompilerParams(use_tc_tiling_on_sc=False,
                                             needs_layout_passes=False),
        scratch_types=[
            pltpu.VMEM((GRP, CHUNK), jnp.int32),
            pltpu.VMEM((GRP, CHUNK), jnp.int32),
            pltpu.VMEM((2, CHUNK, H), jnp.float32),
            pltpu.VMEM((NP2,), jnp.float32),
            pltpu.VMEM_SHARED((NP, H), jnp.float32),
            pltpu.SemaphoreType.DMA,
            pltpu.SemaphoreType.DMA,
        ],
    )
    def seg(xtab_hbm, src_hbm, dst_hbm, sums_hbm, cnt_hbm,
            srcb, dstb, rows, hist, acc_sh, g0, g1):
        c = lax.axis_index("c")
        s = lax.axis_index("s")
        row0 = s * RPT

        zv = jnp.zeros((16,), jnp.float32)
        ov = jnp.ones((16,), jnp.float32)

        @pl.loop(0, CHUNK)
        def _(i):
            @pl.loop(0, H // 16)
            def _(k):
                rows[0, i, pl.ds(k * 16, 16)] = zv

        @pl.loop(0, NP2 // 16)
        def _(i):
            hist[pl.ds(i * 16, 16)] = zv

        @pl.loop(0, RPT // CHUNK)
        def _(r):
            pltpu.sync_copy(rows.at[0], acc_sh.at[pl.ds(row0 + r * CHUNK, CHUNK)])
        rem = RPT - (RPT // CHUNK) * CHUNK
        if rem:
            pltpu.sync_copy(rows.at[0].at[pl.ds(0, rem)],
                            acc_sh.at[pl.ds(row0 + RPT - rem, rem)])

        plsc.subcore_barrier()

        sbase = (c * NTILES + s) * NCH
        dbase = s * NCH

        def gather_start(k, b, sem):
            pltpu.async_copy(xtab_hbm.at[srcb.at[k]], rows.at[b], sem)

        def gather_wait(k, b, sem):
            pltpu.make_async_copy(xtab_hbm.at[srcb.at[k]], rows.at[b], sem).wait()

        def scatter(k, b):
            pltpu.sync_copy(rows.at[b], acc_sh.at[dstb.at[k]], add=True)

        @pl.loop(0, NG)
        def _(g):
            pltpu.sync_copy(src_hbm.at[pl.ds(sbase + g * GRP, GRP)], srcb)
            pltpu.sync_copy(dst_hbm.at[pl.ds(dbase + g * GRP, GRP)], dstb)
            gather_start(0, 0, g0)

            @pl.when(c == 0)
            def _():
                @pl.loop(0, GRP)
                def _(k):
                    @pl.loop(0, CHUNK // 16)
                    def _(i):
                        idxv = dstb[k, pl.ds(i * 16, 16)]
                        plsc.addupdate_scatter(hist, [idxv], ov)

            @pl.loop(0, GRP // 2)
            def _(t):
                k = t * 2
                gather_start(k + 1, 1, g1)
                gather_wait(k, 0, g0)
                scatter(k, 0)

                @pl.when(t < GRP // 2 - 1)
                def _():
                    gather_start(k + 2, 0, g0)

                gather_wait(k + 1, 1, g1)
                scatter(k + 1, 1)

        plsc.subcore_barrier()

        pltpu.sync_copy(acc_sh.at[pl.ds(row0, RPT)],
                        sums_hbm.at[c].at[pl.ds(row0, RPT)])

        @pl.when(c == 0)
        def _():
            pltpu.sync_copy(hist, cnt_hbm.at[s])

    return seg(xtab, src2, dst2)


_BN = 2000


def _tc_cntfold(cnt):
    def body(c_ref, o_ref):
        tot = jnp.sum(c_ref[...], axis=0, keepdims=True)
        r = 1.0 / jnp.maximum(tot, 1.0)
        o_ref[...] = jnp.swapaxes(r, 0, 1)

    return pl.pallas_call(
        body,
        in_specs=[pl.BlockSpec((NTILES, NP2), lambda: (0, 0))],
        out_specs=pl.BlockSpec((NP2, 1), lambda: (0, 0)),
        out_shape=jax.ShapeDtypeStruct((NP2, 1), jnp.float32),
    )(cnt)


def _tc_combine(sums3, rinv, xb, wl0, wl1, wr, b):
    def body(s0_ref, s1_ref, r_ref, x_ref, w0_ref, w1_ref, wr_ref, b_ref,
             o_ref):
        s0 = s0_ref[0]
        s1 = s1_ref[0]
        r = r_ref[...]
        a0 = (s0 * r).astype(jnp.bfloat16)
        a1 = (s1 * r).astype(jnp.bfloat16)
        acc = jnp.dot(a0, w0_ref[...], preferred_element_type=jnp.float32)
        acc = acc + jnp.dot(a1, w1_ref[...], preferred_element_type=jnp.float32)
        acc = acc + jnp.dot(x_ref[...], wr_ref[...],
                            preferred_element_type=jnp.float32)
        o_ref[...] = jnp.maximum(acc + b_ref[...], 0.0)

    return pl.pallas_call(
        body,
        grid=(N // _BN,),
        in_specs=[
            pl.BlockSpec((1, _BN, H), lambda i: (0, i, 0)),
            pl.BlockSpec((1, _BN, H), lambda i: (1, i, 0)),
            pl.BlockSpec((_BN, 1), lambda i: (i, 0)),
            pl.BlockSpec((_BN, D_IN), lambda i: (i, 0)),
            pl.BlockSpec((H, D_OUT), lambda i: (0, 0)),
            pl.BlockSpec((H, D_OUT), lambda i: (0, 0)),
            pl.BlockSpec((D_IN, D_OUT), lambda i: (0, 0)),
            pl.BlockSpec((1, D_OUT), lambda i: (0, 0)),
        ],
        out_specs=pl.BlockSpec((_BN, D_OUT), lambda i: (i, 0)),
        out_shape=jax.ShapeDtypeStruct((N, D_OUT), jnp.float32),
    )(sums3, sums3, rinv, xb, wl0, wl1, wr, b)


def kernel(x, edge_index, W_l, b_l, W_r):
    xtab = x.reshape(2 * N, H)

    src = edge_index[0]
    dst = edge_index[1]
    pad = EPAD - E
    srcp = jnp.concatenate([src, jnp.zeros((pad,), jnp.int32)])
    dstp = jnp.concatenate([dst, jnp.full((pad,), N, jnp.int32)])
    s2 = srcp * 2
    src2 = jnp.concatenate([s2, s2 + 1]).reshape(2 * NTILES * NCH, CHUNK)
    dst2 = dstp.reshape(NTILES * NCH, CHUNK)

    sums, cnt = _sc_segment_sums(xtab, src2, dst2)
    rinv = _tc_cntfold(cnt)

    wlt = W_l.T.astype(jnp.bfloat16)
    out = _tc_combine(sums, rinv, x.astype(jnp.bfloat16),
                      wlt[:H], wlt[H:], W_r.T.astype(jnp.bfloat16),
                      b_l.reshape(1, D_OUT))
    return out
